# baseline (speedup 1.0000x reference)
import jax
import jax.numpy as jnp
from jax.experimental import pallas as pl
from jax.experimental.pallas import tpu as pltpu


def kernel(x, pos_table):
    B, S, D = x.shape
    CR = min(1024, S)
    NS_ = S // CR
    NIT = NS_ * B
    NBUF = 3

    def body(x_hbm, p_hbm, o_hbm, xb, pb, lsem, ssem, psem):
        def s_of(it):
            return it // B

        def b_of(it):
            return it % B

        def xslice(hbm, it):
            return hbm.at[b_of(it), pl.ds(s_of(it) * CR, CR), :]

        pltpu.make_async_copy(
            p_hbm.at[pl.ds(0, CR)], pb.at[0], psem.at[0]).start()
        if NS_ > 1:
            pltpu.make_async_copy(
                p_hbm.at[pl.ds(CR, CR)], pb.at[1], psem.at[1]).start()
        pltpu.make_async_copy(xslice(x_hbm, 0), xb.at[0], lsem.at[0]).start()

        for it in range(NIT):
            sl = it % NBUF
            if it + 1 < NIT:
                nsl = (it + 1) % NBUF
                if it + 1 >= NBUF:
                    pltpu.make_async_copy(
                        xb.at[nsl], xslice(o_hbm, it + 1 - NBUF),
                        ssem.at[nsl]).wait()
                pltpu.make_async_copy(
                    xslice(x_hbm, it + 1), xb.at[nsl], lsem.at[nsl]).start()
            pltpu.make_async_copy(xslice(x_hbm, it), xb.at[sl], lsem.at[sl]).wait()
            si = s_of(it)
            if b_of(it) == 0:
                pltpu.make_async_copy(
                    p_hbm.at[pl.ds(si * CR, CR)], pb.at[si % 2], psem.at[si % 2]).wait()
            xb[sl] = xb[sl] + pb[si % 2]
            if b_of(it) == B - 1 and si + 2 < NS_:
                pltpu.make_async_copy(
                    p_hbm.at[pl.ds((si + 2) * CR, CR)], pb.at[si % 2],
                    psem.at[si % 2]).start()
            pltpu.make_async_copy(xb.at[sl], xslice(o_hbm, it), ssem.at[sl]).start()

        for it in range(max(0, NIT - NBUF), NIT):
            pltpu.make_async_copy(
                xb.at[it % NBUF], xslice(o_hbm, it), ssem.at[it % NBUF]).wait()

    return pl.pallas_call(
        body,
        in_specs=[
            pl.BlockSpec(memory_space=pltpu.MemorySpace.HBM),
            pl.BlockSpec(memory_space=pltpu.MemorySpace.HBM),
        ],
        out_specs=pl.BlockSpec(memory_space=pltpu.MemorySpace.HBM),
        out_shape=jax.ShapeDtypeStruct((B, S, D), x.dtype),
        scratch_shapes=[
            pltpu.VMEM((NBUF, CR, D), jnp.float32),
            pltpu.VMEM((2, CR, D), jnp.float32),
            pltpu.SemaphoreType.DMA((NBUF,)),
            pltpu.SemaphoreType.DMA((NBUF,)),
            pltpu.SemaphoreType.DMA((2,)),
        ],
    )(x, pos_table[:S])
